# baseline (speedup 1.0000x reference)
import jax
import jax.numpy as jnp
from jax import lax
from jax.experimental import pallas as pl
from jax.experimental.pallas import tpu as pltpu
from jax.experimental.pallas import tpu_sc as plsc

BATCH = 4096
TOKENS = 200
DIM = 2048
ROWS = BATCH * TOKENS

NUM_CORES = 2
NUM_SUBCORES = 16
NUM_WORKERS = NUM_CORES * NUM_SUBCORES

NPART = 4
PART_ROWS = TOKENS // NPART
NGROUPS = NUM_WORKERS // NPART
PER_GROUP = ROWS // NGROUPS
IBLK = 2048
NBLK = PER_GROUP // IBLK
NVEC = IBLK // 16


def _sc_body(idx_hbm, table_hbm, out_hbm, table_v, idx_v, pos_v, lrow_v,
             isem, wsem):
    wid = lax.axis_index("s") * NUM_CORES + lax.axis_index("c")
    grp = wid // NPART
    part = wid % NPART
    lo = part * PART_ROWS
    gbase = grp * PER_GROUP

    pltpu.sync_copy(table_hbm.at[pl.ds(lo * DIM, PART_ROWS * DIM)], table_v)

    def start_idx(blk, b):
        pltpu.async_copy(
            idx_hbm.at[pl.ds(gbase + blk * IBLK, IBLK)], idx_v.at[b],
            isem.at[b],
        )

    def wait_idx(b):
        pltpu.make_async_copy(
            idx_hbm.at[pl.ds(0, IBLK)], idx_v.at[b], isem.at[b]
        ).wait()

    def wait_row_write():
        pltpu.make_async_copy(
            table_v.at[pl.ds(0, DIM)], out_hbm.at[pl.ds(0, DIM)], wsem
        ).wait()

    start_idx(0, 0)
    lanes = lax.iota(jnp.int32, 16)

    def do_block(blk, b, m_prev):
        wait_idx(b)

        @pl.when(blk + 1 < NBLK)
        def _():
            start_idx(blk + 1, 1 - b)

        def scan_step(i, m):
            v = idx_v[b, pl.ds(i * 16, 16)]
            rel = v - jnp.broadcast_to(lo, (16,))
            msk = (rel >= jnp.broadcast_to(0, (16,))) & (
                rel < jnp.broadcast_to(PART_ROWS, (16,)))
            hits = msk.astype(jnp.int32)
            csum = plsc.cumsum(hits)
            dest = jnp.broadcast_to(m, (16,)) + csum - hits
            pos = jnp.broadcast_to(blk * IBLK + i * 16, (16,)) + lanes
            plsc.store_scatter(pos_v, [dest], pos, mask=msk)
            plsc.store_scatter(lrow_v, [dest], rel, mask=msk)
            return m + jnp.max(csum)

        m = lax.fori_loop(0, NVEC, scan_step, jnp.int32(0))

        @pl.loop(0, (m + 15) // 16)
        def _fire(g):
            k16 = g * 16
            pv = pos_v[pl.ds(pl.multiple_of(k16, 16), 16)]
            rv = lrow_v[pl.ds(pl.multiple_of(k16, 16), 16)]
            for l in range(16):
                @pl.when(k16 + l < m)
                def _():
                    src = pl.multiple_of(rv[l] * DIM, DIM)
                    dst = pl.multiple_of((gbase + pv[l]) * DIM, DIM)
                    pltpu.async_copy(
                        table_v.at[pl.ds(src, DIM)],
                        out_hbm.at[pl.ds(dst, DIM)],
                        wsem,
                    )

        @pl.loop(0, m_prev)
        def _drain(k):
            wait_row_write()

        return m

    def block_pair(j2, m_prev):
        m_prev = do_block(j2 * 2, 0, m_prev)
        return do_block(j2 * 2 + 1, 1, m_prev)

    m_last = lax.fori_loop(0, NBLK // 2, block_pair, jnp.int32(0))

    @pl.loop(0, m_last)
    def _final_drain(k):
        wait_row_write()


@jax.jit
def _sc_gather(idx_flat, table_flat):
    mesh = plsc.VectorSubcoreMesh(
        core_axis_name="c", subcore_axis_name="s",
        num_cores=NUM_CORES, num_subcores=NUM_SUBCORES,
    )
    call = pl.kernel(
        _sc_body,
        out_type=jax.ShapeDtypeStruct((ROWS * DIM,), jnp.float32),
        mesh=mesh,
        compiler_params=pltpu.CompilerParams(needs_layout_passes=False),
        scratch_types=[
            pltpu.VMEM((PART_ROWS * DIM,), jnp.float32),
            pltpu.VMEM((2, IBLK), jnp.int32),
            pltpu.VMEM((IBLK + 16,), jnp.int32),
            pltpu.VMEM((IBLK + 16,), jnp.int32),
            pltpu.SemaphoreType.DMA((2,)),
            pltpu.SemaphoreType.DMA,
        ],
    )
    return call(idx_flat, table_flat)


def kernel(indices, embeddings):
    idx_flat = indices.reshape(ROWS).astype(jnp.int32)
    out = _sc_gather(idx_flat, embeddings.reshape(TOKENS * DIM))
    return out.reshape(BATCH, TOKENS, DIM)
